# baseline (speedup 1.0000x reference)
import functools

import jax
import jax.numpy as jnp
from jax import lax
from jax.experimental import pallas as pl
from jax.experimental.pallas import tpu as pltpu
from jax.experimental.pallas import tpu_sc as plsc

_NC = 2
_NS = 16

_B = 125


def _mesh(nc):
    return plsc.VectorSubcoreMesh(core_axis_name="c", subcore_axis_name="s",
                                  num_cores=nc)


def _scatter_loop(gather_ref, raw_ref, cid, dst_v, rows_v, sidx, acc_sh,
                  sems, nb):
    def xform(j, buf):
        base = j * _B
        for k in range(-(-_B // 16)):
            off = min(16 * k, _B - 16)
            v = raw_ref[pl.ds(base + off, 16)]
            sidx[buf, pl.ds(off, 16)] = v * 2 + cid

    for j in range(3):
        xform(j, j)
        pltpu.async_copy(gather_ref.at[sidx.at[j]], rows_v.at[j], sems[j])

    def step4(g, carry):
        for b in range(4):
            j = 4 * g + b
            nbuf = (b + 3) % 4
            xform(j + 3, nbuf)
            pltpu.async_copy(gather_ref.at[sidx.at[nbuf]], rows_v.at[nbuf],
                             sems[nbuf])
            pltpu.make_async_copy(gather_ref.at[sidx.at[b]], rows_v.at[b],
                                  sems[b]).wait()
            pltpu.sync_copy(rows_v.at[b], acc_sh.at[dst_v.at[j]], add=True)
        return carry

    lax.fori_loop(0, (nb - 4) // 4, step4, 0)
    for b in range(4):
        j = nb - 4 + b
        if b == 0:
            xform(nb - 1, 3)
            pltpu.async_copy(gather_ref.at[sidx.at[3]], rows_v.at[3], sems[3])
        pltpu.make_async_copy(gather_ref.at[sidx.at[b]], rows_v.at[b],
                              sems[b]).wait()
        pltpu.sync_copy(rows_v.at[b], acc_sh.at[dst_v.at[j]], add=True)


def _sc_degree(dst3, n):
    nb = dst3.shape[1]
    rps = n // _NS

    @functools.partial(
        pl.kernel,
        out_type=jax.ShapeDtypeStruct((n, 16), jnp.float32),
        mesh=_mesh(1),
        compiler_params=pltpu.CompilerParams(use_tc_tiling_on_sc=False),
        scratch_types=[
            pltpu.VMEM((nb, _B), jnp.int32),
            pltpu.VMEM((_B, 16), jnp.float32),
            pltpu.VMEM_SHARED((n, 16), jnp.float32),
        ],
    )
    def k(dst_hbm, out_hbm, dst_v, ones_v, acc_sh):
        sid = lax.axis_index("s")
        r0 = sid * rps

        def ostep(i, carry):
            ones_v[i, pl.ds(0, 16)] = jnp.ones((16,), jnp.float32)
            return carry

        lax.fori_loop(0, _B, ostep, 0)
        for t in range(rps // _B):
            pltpu.sync_copy(ones_v, acc_sh.at[pl.ds(r0 + t * _B, _B)])
        pltpu.sync_copy(dst_hbm.at[sid], dst_v)
        plsc.subcore_barrier()

        def step(j, carry):
            pltpu.sync_copy(ones_v, acc_sh.at[dst_v.at[j]], add=True)
            return carry

        lax.fori_loop(0, nb, step, 0)
        plsc.subcore_barrier()
        pltpu.sync_copy(acc_sh.at[pl.ds(r0, rps)], out_hbm.at[pl.ds(r0, rps)])

    return k(dst3)


def _sc_aggregate(hs_view, src_flat, dst3, split_out):
    n2, dh = hs_view.shape
    n = n2 // 2
    nb = src_flat.shape[1] // _B
    rps = n // _NS
    out_t = (jax.ShapeDtypeStruct((_NC, n, dh), jnp.float32) if split_out
             else jax.ShapeDtypeStruct((n, 2 * dh), jnp.float32))

    @functools.partial(
        pl.kernel,
        out_type=out_t,
        mesh=_mesh(_NC),
        compiler_params=pltpu.CompilerParams(use_tc_tiling_on_sc=False),
        scratch_types=[
            pltpu.VMEM((nb * _B,), jnp.int32),
            pltpu.VMEM((4, _B), jnp.int32),
            pltpu.VMEM((nb, _B), jnp.int32),
            pltpu.VMEM((4, _B, dh), jnp.float32),
            pltpu.VMEM_SHARED((n, dh), jnp.float32),
            [pltpu.SemaphoreType.DMA] * 4,
        ],
    )
    def k(hs_hbm, src_hbm, dst_hbm, out_hbm, raw_v, sidx_v, dst_v, rows_v,
          acc_sh, sems):
        cid = lax.axis_index("c")
        sid = lax.axis_index("s")
        r0 = sid * rps
        def zstep(i, carry):
            for q in range(-(-dh // 16)):
                off = min(q * 16, dh - 16)
                rows_v[0, i, pl.ds(off, 16)] = jnp.zeros((16,), jnp.float32)
            return carry

        lax.fori_loop(0, _B, zstep, 0)
        assert rps % _B == 0
        for t in range(rps // _B):
            pltpu.sync_copy(rows_v.at[0], acc_sh.at[pl.ds(r0 + t * _B, _B)])
        pltpu.sync_copy(src_hbm.at[sid], raw_v)
        pltpu.sync_copy(dst_hbm.at[sid], dst_v)
        plsc.subcore_barrier()
        _scatter_loop(hs_hbm, raw_v, cid, dst_v, rows_v, sidx_v, acc_sh,
                      sems, nb)
        plsc.subcore_barrier()
        if split_out:
            pltpu.sync_copy(acc_sh.at[pl.ds(r0, rps)],
                            out_hbm.at[cid, pl.ds(r0, rps)])
        else:
            pltpu.sync_copy(acc_sh.at[pl.ds(r0, rps)],
                            out_hbm.at[pl.ds(r0, rps), pl.ds(cid * dh, dh)])

    return k(hs_view, src_flat, dst3)


def _dinv_from(deg_ref):
    return lax.rsqrt(deg_ref[:, :1])


def _tc_matmul(x, w1):
    def body(x_ref, w1_ref, out_ref):
        out_ref[...] = jnp.dot(x_ref[...], w1_ref[...],
                               preferred_element_type=jnp.float32,
                               precision=lax.Precision.HIGHEST)

    return pl.pallas_call(
        body,
        out_shape=jax.ShapeDtypeStruct((x.shape[0], w1.shape[1]),
                                       jnp.float32),
    )(x, w1)


def _tc_scale(xw, deg):
    def body(xw_ref, deg_ref, out_ref):
        out_ref[...] = xw_ref[...] * _dinv_from(deg_ref)

    return pl.pallas_call(
        body,
        out_shape=jax.ShapeDtypeStruct(xw.shape, jnp.float32),
    )(xw, deg)


def _tc_bn_relu_matmul(p, hs, deg, gamma, beta, b1, w2p):
    n = hs.shape[0]
    d2 = w2p.shape[1]

    def body(p_ref, hs_ref, deg_ref, g_ref, be_ref, b1_ref, w2_ref, out_ref):
        dinv = _dinv_from(deg_ref)
        agg = p_ref[...] + hs_ref[...]
        h = agg * dinv + b1_ref[...][None, :]
        mean = jnp.mean(h, axis=0)
        var = jnp.mean((h - mean[None, :]) ** 2, axis=0)
        hn = (h - mean[None, :]) / jnp.sqrt(var + 1e-5)[None, :]
        hr = jnp.maximum(g_ref[...][None, :] * hn + be_ref[...][None, :], 0.0)
        h2 = jnp.dot(hr, w2_ref[...],
                     preferred_element_type=jnp.float32,
                     precision=lax.Precision.HIGHEST)
        out_ref[...] = h2 * dinv

    return pl.pallas_call(
        body,
        out_shape=jax.ShapeDtypeStruct((n, d2), jnp.float32),
    )(p, hs, deg, gamma, beta, b1, w2p)


def _tc_finish(p2, hs2, deg, b2, d_out):
    def body(p_ref, hs_ref, deg_ref, b2_ref, out_ref):
        agg = p_ref[...] + hs_ref[...]
        sc = agg * _dinv_from(deg_ref)
        out_ref[...] = sc[:, :d_out] + b2_ref[...][None, :]

    return pl.pallas_call(
        body,
        out_shape=jax.ShapeDtypeStruct((hs2.shape[0], d_out), jnp.float32),
    )(p2, hs2, deg, b2)


def kernel(x, adj_t, W1, b1, gamma1, beta1, W2, b2):
    n = x.shape[0]
    e = adj_t.shape[1]
    ept = e // _NS
    nb = ept // _B
    assert ept == nb * _B and n % _NS == 0

    src_flat = adj_t[0].astype(jnp.int32).reshape(_NS, nb * _B)
    dst3 = adj_t[1].astype(jnp.int32).reshape(_NS, nb, _B)

    deg = _sc_degree(dst3, n)

    xw = _tc_matmul(x, W1)
    hs = _tc_scale(xw, deg)
    p1 = _sc_aggregate(hs.reshape(2 * n, -1), src_flat, dst3,
                       split_out=False)

    d2pad = 48
    w2p = jnp.zeros((W2.shape[0], d2pad), jnp.float32).at[:, :W2.shape[1]].set(W2)

    hs2 = _tc_bn_relu_matmul(p1, hs, deg, gamma1, beta1, b1, w2p)
    p2 = _sc_aggregate(hs2.reshape(2 * n, -1), src_flat, dst3,
                       split_out=False)
    return _tc_finish(p2, hs2, deg, b2, W2.shape[1])


# speedup vs baseline: 1.0125x; 1.0125x over previous
import functools

import jax
import jax.numpy as jnp
from jax import lax
from jax.experimental import pallas as pl
from jax.experimental.pallas import tpu as pltpu
from jax.experimental.pallas import tpu_sc as plsc

_NC = 2
_NS = 16

_B = 125


def _mesh(nc):
    return plsc.VectorSubcoreMesh(core_axis_name="c", subcore_axis_name="s",
                                  num_cores=nc)


def _scatter_loop(gref, src_v, dst_v, rows_v, acc_sh, gsems, ssems, nb):
    for j in range(3):
        pltpu.async_copy(gref.at[src_v.at[j]], rows_v.at[j], gsems[j])

    def step4(g, carry):
        for b in range(4):
            j = 4 * g + b
            pltpu.async_copy(gref.at[src_v.at[j + 3]],
                             rows_v.at[(b + 3) % 4], gsems[(b + 3) % 4])
            pltpu.make_async_copy(gref.at[src_v.at[j]], rows_v.at[b],
                                  gsems[b]).wait()
            pltpu.sync_copy(rows_v.at[b], acc_sh.at[dst_v.at[j]], add=True)
        return carry

    lax.fori_loop(0, (nb - 4) // 4, step4, 0)
    for b in range(4):
        j = nb - 4 + b
        if b == 0:
            pltpu.async_copy(gref.at[src_v.at[nb - 1]], rows_v.at[3],
                             gsems[3])
        pltpu.make_async_copy(gref.at[src_v.at[j]], rows_v.at[b],
                              gsems[b]).wait()
        pltpu.sync_copy(rows_v.at[b], acc_sh.at[dst_v.at[j]], add=True)


def _sc_degree(dst3, n):
    nb = dst3.shape[1]
    rps = n // _NS

    @functools.partial(
        pl.kernel,
        out_type=jax.ShapeDtypeStruct((n, 16), jnp.float32),
        mesh=_mesh(1),
        compiler_params=pltpu.CompilerParams(use_tc_tiling_on_sc=False),
        scratch_types=[
            pltpu.VMEM((nb, _B), jnp.int32),
            pltpu.VMEM((_B, 16), jnp.float32),
            pltpu.VMEM_SHARED((n, 16), jnp.float32),
        ],
    )
    def k(dst_hbm, out_hbm, dst_v, ones_v, acc_sh):
        sid = lax.axis_index("s")
        r0 = sid * rps

        def ostep(i, carry):
            ones_v[i, pl.ds(0, 16)] = jnp.ones((16,), jnp.float32)
            return carry

        lax.fori_loop(0, _B, ostep, 0)
        for t in range(rps // _B):
            pltpu.sync_copy(ones_v, acc_sh.at[pl.ds(r0 + t * _B, _B)])
        pltpu.sync_copy(dst_hbm.at[sid], dst_v)
        plsc.subcore_barrier()

        def step(j, carry):
            pltpu.sync_copy(ones_v, acc_sh.at[dst_v.at[j]], add=True)
            return carry

        lax.fori_loop(0, nb, step, 0)
        plsc.subcore_barrier()
        pltpu.sync_copy(acc_sh.at[pl.ds(r0, rps)], out_hbm.at[pl.ds(r0, rps)])

    return k(dst3)


def _sc_aggregate(hs_view, src4, dst3, split_out):
    n2, dh = hs_view.shape
    n = n2 // 2
    nb = src4.shape[2]
    rps = n // _NS
    out_t = (jax.ShapeDtypeStruct((_NC, n, dh), jnp.float32) if split_out
             else jax.ShapeDtypeStruct((n, 2 * dh), jnp.float32))

    @functools.partial(
        pl.kernel,
        out_type=out_t,
        mesh=_mesh(_NC),
        compiler_params=pltpu.CompilerParams(use_tc_tiling_on_sc=False),
        scratch_types=[
            pltpu.VMEM((nb, _B), jnp.int32),
            pltpu.VMEM((nb, _B), jnp.int32),
            pltpu.VMEM((4, _B, dh), jnp.float32),
            pltpu.VMEM_SHARED((n, dh), jnp.float32),
            [pltpu.SemaphoreType.DMA] * 4,
            [pltpu.SemaphoreType.DMA] * 4,
        ],
    )
    def k(hs_hbm, src_hbm, dst_hbm, out_hbm, src_v, dst_v, rows_v,
          acc_sh, gsems, ssems):
        cid = lax.axis_index("c")
        sid = lax.axis_index("s")
        r0 = sid * rps
        def zstep(i, carry):
            for q in range(-(-dh // 16)):
                off = min(q * 16, dh - 16)
                rows_v[0, i, pl.ds(off, 16)] = jnp.zeros((16,), jnp.float32)
            return carry

        lax.fori_loop(0, _B, zstep, 0)
        assert rps % _B == 0 and nb % 4 == 0
        for t in range(rps // _B):
            pltpu.sync_copy(rows_v.at[0], acc_sh.at[pl.ds(r0 + t * _B, _B)])
        pltpu.sync_copy(src_hbm.at[cid, sid], src_v)
        pltpu.sync_copy(dst_hbm.at[sid], dst_v)
        plsc.subcore_barrier()
        _scatter_loop(hs_hbm, src_v, dst_v, rows_v, acc_sh, gsems, ssems, nb)
        plsc.subcore_barrier()
        if split_out:
            pltpu.sync_copy(acc_sh.at[pl.ds(r0, rps)],
                            out_hbm.at[cid, pl.ds(r0, rps)])
        else:
            pltpu.sync_copy(acc_sh.at[pl.ds(r0, rps)],
                            out_hbm.at[pl.ds(r0, rps), pl.ds(cid * dh, dh)])

    return k(hs_view, src4, dst3)


def _dinv_from(deg_ref):
    return lax.rsqrt(deg_ref[:, :1])


def _tc_matmul(x, w1):
    def body(x_ref, w1_ref, out_ref):
        out_ref[...] = jnp.dot(x_ref[...], w1_ref[...],
                               preferred_element_type=jnp.float32,
                               precision=lax.Precision.HIGHEST)

    return pl.pallas_call(
        body,
        out_shape=jax.ShapeDtypeStruct((x.shape[0], w1.shape[1]),
                                       jnp.float32),
    )(x, w1)


def _tc_scale(xw, deg):
    def body(xw_ref, deg_ref, out_ref):
        out_ref[...] = xw_ref[...] * _dinv_from(deg_ref)

    return pl.pallas_call(
        body,
        out_shape=jax.ShapeDtypeStruct(xw.shape, jnp.float32),
    )(xw, deg)


def _tc_bn_relu_matmul(p, hs, deg, gamma, beta, b1, w2p):
    n = hs.shape[0]
    d2 = w2p.shape[1]

    def body(p_ref, hs_ref, deg_ref, g_ref, be_ref, b1_ref, w2_ref, out_ref):
        dinv = _dinv_from(deg_ref)
        agg = p_ref[...] + hs_ref[...]
        h = agg * dinv + b1_ref[...][None, :]
        mean = jnp.mean(h, axis=0)
        var = jnp.mean((h - mean[None, :]) ** 2, axis=0)
        hn = (h - mean[None, :]) / jnp.sqrt(var + 1e-5)[None, :]
        hr = jnp.maximum(g_ref[...][None, :] * hn + be_ref[...][None, :], 0.0)
        h2 = jnp.dot(hr, w2_ref[...],
                     preferred_element_type=jnp.float32,
                     precision=lax.Precision.HIGHEST)
        out_ref[...] = h2 * dinv

    return pl.pallas_call(
        body,
        out_shape=jax.ShapeDtypeStruct((n, d2), jnp.float32),
    )(p, hs, deg, gamma, beta, b1, w2p)


def _tc_finish(p2, hs2, deg, b2, d_out):
    def body(p_ref, hs_ref, deg_ref, b2_ref, out_ref):
        agg = p_ref[...] + hs_ref[...]
        sc = agg * _dinv_from(deg_ref)
        out_ref[...] = sc[:, :d_out] + b2_ref[...][None, :]

    return pl.pallas_call(
        body,
        out_shape=jax.ShapeDtypeStruct((hs2.shape[0], d_out), jnp.float32),
    )(p2, hs2, deg, b2)


def kernel(x, adj_t, W1, b1, gamma1, beta1, W2, b2):
    n = x.shape[0]
    e = adj_t.shape[1]
    ept = e // _NS
    nb = ept // _B
    assert ept == nb * _B and n % _NS == 0

    src3 = adj_t[0].astype(jnp.int32).reshape(_NS, nb, _B)
    dst3 = adj_t[1].astype(jnp.int32).reshape(_NS, nb, _B)
    src4 = jnp.stack([2 * src3, 2 * src3 + 1], axis=0)

    deg = _sc_degree(dst3, n)

    xw = _tc_matmul(x, W1)
    hs = _tc_scale(xw, deg)
    p1 = _sc_aggregate(hs.reshape(2 * n, -1), src4, dst3,
                       split_out=False)

    d2pad = 48
    w2p = jnp.zeros((W2.shape[0], d2pad), jnp.float32).at[:, :W2.shape[1]].set(W2)

    hs2 = _tc_bn_relu_matmul(p1, hs, deg, gamma1, beta1, b1, w2p)
    p2 = _sc_aggregate(hs2.reshape(2 * n, -1), src4, dst3,
                       split_out=False)
    return _tc_finish(p2, hs2, deg, b2, W2.shape[1])
